# baseline (speedup 1.0000x reference)
import functools

import jax
import jax.numpy as jnp
from jax import lax
from jax.experimental import pallas as pl
from jax.experimental.pallas import tpu as pltpu
from jax.experimental.pallas import tpu_sc as plsc

_SC_ROWS = 1024


def _tc_copy_block(t_ref, o_ref):
    o_ref[...] = t_ref[...]


def kernel(x, table):
    seq = x.shape[1]
    emb = table.shape[1]
    info = plsc.get_sparse_core_info()
    nw = info.num_cores * info.num_subcores
    sc_rows = _SC_ROWS
    rows_per_w = sc_rows // nw
    mesh = plsc.VectorSubcoreMesh(core_axis_name="c", subcore_axis_name="s")

    @functools.partial(
        pl.kernel,
        out_type=jax.ShapeDtypeStruct((sc_rows, emb), table.dtype),
        mesh=mesh,
        scratch_types=[
            pltpu.VMEM((rows_per_w, emb), jnp.float32),
            pltpu.SemaphoreType.DMA,
        ],
    )
    def sc_copy(table_hbm, out_hbm, buf, sem):
        wid = lax.axis_index("s") * info.num_cores + lax.axis_index("c")
        base = wid * rows_per_w
        pltpu.make_async_copy(table_hbm.at[pl.ds(base, rows_per_w)], buf, sem).start()
        pltpu.make_async_copy(table_hbm.at[pl.ds(base, rows_per_w)], buf, sem).wait()
        pltpu.make_async_copy(buf, out_hbm.at[pl.ds(base, rows_per_w)], sem).start()
        pltpu.make_async_copy(buf, out_hbm.at[pl.ds(base, rows_per_w)], sem).wait()

    sc_part = sc_copy(table)

    tc_rows = seq - sc_rows
    block = 1024
    skip = sc_rows // block
    tc_out = pl.pallas_call(
        _tc_copy_block,
        grid=(tc_rows // block,),
        in_specs=[pl.BlockSpec((block, emb), lambda i: (i + skip, 0))],
        out_specs=pl.BlockSpec((block, emb), lambda i: (i + skip, 0)),
        out_shape=jax.ShapeDtypeStruct((seq, emb), table.dtype),
    )(table)

    out = lax.dynamic_update_slice(tc_out, sc_part, (0, 0))
    return out[None, :, :]
